# baseline (speedup 1.0000x reference)
import functools

import jax
import jax.numpy as jnp
from jax import lax
from jax.experimental import pallas as pl
from jax.experimental.pallas import tpu as pltpu
from jax.experimental.pallas import tpu_sc as plsc

_GAMMA = 0.05
_D = 512
_C = 10
_B = 16384

_NC = 1
_NS = 16
_L = 16
_NW = _NC * _NS
_PER_W = _B // _NW


def _lse_body(w_ref, lse_ref):
  w = w_ref[...]
  g = lax.dot_general(w, w, (((0,), (0,)), ((), ())),
                      preferred_element_type=jnp.float32)
  ii = lax.broadcasted_iota(jnp.int32, (_C, _C), 0)
  jj = lax.broadcasted_iota(jnp.int32, (_C, _C), 1)
  e = jnp.where(ii == jj, 0.0, jnp.exp(g * _GAMMA))
  lse = jnp.log(jnp.sum(e, axis=1, keepdims=True)) * (1.0 / _GAMMA)
  bcast = jnp.broadcast_to(lse, (_C, _L))
  lse_ref[...] = jnp.concatenate(
      [bcast, jnp.zeros((_L - _C, _L), jnp.float32)], axis=0)


_lse = pl.pallas_call(
    _lse_body,
    out_shape=jax.ShapeDtypeStruct((_L, _L), jnp.float32),
)


def _hist_body(t_hbm, lse_hbm, out_hbm, t_v, lse_v, row_v, grid_v, rows_v,
               dup_v):
  wid = lax.axis_index("s") * _NC + lax.axis_index("c")
  pltpu.sync_copy(t_hbm.at[pl.ds(wid * _PER_W, _PER_W)], t_v)
  pltpu.sync_copy(lse_hbm, lse_v)

  zero = jnp.zeros((_L,), jnp.int32)
  one = jnp.ones((_L,), jnp.int32)

  def step(i, accs):
    t = t_v[pl.ds(i * _L, _L)]
    return tuple(a + jnp.where(t == c, one, zero) for c, a in enumerate(accs))

  accs = lax.fori_loop(0, _PER_W // _L, step, (zero,) * _C)

  pvec = jnp.zeros((_L,), jnp.float32)
  for c in range(_C):
    pvec = pvec + accs[c].astype(jnp.float32) * lse_v[c]
  row_v[...] = pvec * (1.0 / _B)
  pltpu.sync_copy(row_v, out_hbm.at[wid])


@functools.cache
def _hist():
  return functools.partial(
      pl.kernel,
      mesh=plsc.VectorSubcoreMesh(
          core_axis_name="c", subcore_axis_name="s",
          num_cores=_NC, num_subcores=_NS),
      out_type=jax.ShapeDtypeStruct((_NW, _L), jnp.float32),
      scratch_types=[
          pltpu.VMEM((_PER_W,), jnp.int32),
          pltpu.VMEM((_L, _L), jnp.float32),
          pltpu.VMEM((_L,), jnp.float32),
          pltpu.VMEM_SHARED((_NW, _L), jnp.float32),
          pltpu.VMEM((_NW, _L), jnp.float32),
          pltpu.VMEM((2 * _L,), jnp.float32),
      ],
  )(_hist_body)


@jax.jit
def kernel(weight, target):
  lse = _lse(weight)
  out = _hist()(target, lse)
  return jnp.sum(out)


# speedup vs baseline: 1.0761x; 1.0761x over previous
import functools

import jax
import jax.numpy as jnp
from jax import lax
from jax.experimental import pallas as pl
from jax.experimental.pallas import tpu as pltpu
from jax.experimental.pallas import tpu_sc as plsc

_GAMMA = 0.05
_D = 512
_C = 10
_B = 16384

_NC = 1
_NS = 16
_L = 16
_NW = _NC * _NS
_PER_W = _B // _NW


def _hist_body(t_hbm, out_hbm, t_v, acc_v):
  wid = lax.axis_index("s") * _NC + lax.axis_index("c")
  pltpu.sync_copy(t_hbm.at[pl.ds(wid * _PER_W, _PER_W)], t_v)

  zero = jnp.zeros((_L,), jnp.int32)
  one = jnp.ones((_L,), jnp.int32)

  def step(i, carry):
    lo, hi = carry
    t = t_v[pl.ds(i * _L, _L)]
    m = t < 5
    sh = t * 6
    lo = lo + lax.shift_left(jnp.where(m, one, zero), jnp.where(m, sh, zero))
    hi = hi + lax.shift_left(jnp.where(m, zero, one),
                             jnp.where(m, zero, sh - 30))
    return lo, hi

  n_vregs = _PER_W // _L
  accs = [zero] * _C
  for half in range(2):
    lo, hi = lax.fori_loop(half * (n_vregs // 2), (half + 1) * (n_vregs // 2),
                           step, (zero, zero))
    for c in range(5):
      accs[c] = accs[c] + ((lo >> (6 * c)) & 63)
      accs[c + 5] = accs[c + 5] + ((hi >> (6 * c)) & 63)
  for c in range(_C):
    acc_v[c, :] = accs[c].astype(jnp.float32)

  pltpu.sync_copy(acc_v, out_hbm.at[wid])


@functools.cache
def _hist():
  return functools.partial(
      pl.kernel,
      mesh=plsc.VectorSubcoreMesh(
          core_axis_name="c", subcore_axis_name="s",
          num_cores=_NC, num_subcores=_NS),
      out_type=jax.ShapeDtypeStruct((_NW, _C, _L), jnp.float32),
      scratch_types=[
          pltpu.VMEM((_PER_W,), jnp.int32),
          pltpu.VMEM((_C, _L), jnp.float32),
      ],
  )(_hist_body)


def _lse_body(w_ref, lse_ref):
  w = w_ref[...]
  g = lax.dot_general(w, w, (((0,), (0,)), ((), ())),
                      preferred_element_type=jnp.float32)
  ii = lax.broadcasted_iota(jnp.int32, (_C, _C), 0)
  jj = lax.broadcasted_iota(jnp.int32, (_C, _C), 1)
  e = jnp.where(ii == jj, 0.0, jnp.exp(g * _GAMMA))
  lse_ref[...] = jnp.log(jnp.sum(e, axis=0, keepdims=True)) * (1.0 / _GAMMA)


_lse = pl.pallas_call(
    _lse_body,
    out_shape=jax.ShapeDtypeStruct((1, _C), jnp.float32),
)


def _combine_body(lse_ref, p_ref, out_ref):
  lse = lse_ref[...]
  p = p_ref[...]
  counts = jnp.sum(jnp.sum(p, axis=2), axis=0, keepdims=True)
  out_ref[...] = jnp.sum(counts * lse, axis=1, keepdims=True) * (1.0 / _B)


_combine = pl.pallas_call(
    _combine_body,
    out_shape=jax.ShapeDtypeStruct((1, 1), jnp.float32),
)


@jax.jit
def kernel(weight, target):
  partials = _hist()(target)
  lse = _lse(weight)
  out = _combine(lse, partials)
  return out[0, 0]


# speedup vs baseline: 1.0777x; 1.0014x over previous
import functools

import jax
import jax.numpy as jnp
from jax import lax
from jax.experimental import pallas as pl
from jax.experimental.pallas import tpu as pltpu
from jax.experimental.pallas import tpu_sc as plsc

_GAMMA = 0.05
_D = 512
_C = 10
_B = 16384

_NC = 1
_NS = 16
_L = 16
_NW = _NC * _NS
_PER_W = _B // _NW


def _hist_body(t_hbm, out_hbm, t_v, acc_v):
  wid = lax.axis_index("s") * _NC + lax.axis_index("c")
  pltpu.sync_copy(t_hbm.at[pl.ds(wid * _PER_W, _PER_W)], t_v)

  zero = jnp.zeros((_L,), jnp.int32)
  one = jnp.ones((_L,), jnp.int32)

  def step(i, accs):
    t = t_v[pl.ds(i * _L, _L)]
    return tuple(a + jnp.where(t == c, one, zero) for c, a in enumerate(accs))

  accs = lax.fori_loop(0, _PER_W // _L, step, (zero,) * _C)
  for c in range(_C):
    acc_v[c, :] = accs[c].astype(jnp.float32)

  pltpu.sync_copy(acc_v, out_hbm.at[wid])


@functools.cache
def _hist():
  return functools.partial(
      pl.kernel,
      mesh=plsc.VectorSubcoreMesh(
          core_axis_name="c", subcore_axis_name="s",
          num_cores=_NC, num_subcores=_NS),
      out_type=jax.ShapeDtypeStruct((_NW, _C, _L), jnp.float32),
      scratch_types=[
          pltpu.VMEM((_PER_W,), jnp.int32),
          pltpu.VMEM((_C, _L), jnp.float32),
      ],
  )(_hist_body)


def _lse_body(w_ref, lse_ref):
  w = w_ref[...]
  g = lax.dot_general(w, w, (((0,), (0,)), ((), ())),
                      preferred_element_type=jnp.float32)
  ii = lax.broadcasted_iota(jnp.int32, (_C, _C), 0)
  jj = lax.broadcasted_iota(jnp.int32, (_C, _C), 1)
  e = jnp.where(ii == jj, 0.0, jnp.exp(g * _GAMMA))
  lse_ref[...] = jnp.log(jnp.sum(e, axis=0, keepdims=True)) * (1.0 / _GAMMA)


_lse = pl.pallas_call(
    _lse_body,
    out_shape=jax.ShapeDtypeStruct((1, _C), jnp.float32),
)


def _combine_body(lse_ref, p_ref, out_ref):
  lse = lse_ref[...]
  p = p_ref[...]
  counts = jnp.sum(jnp.sum(p, axis=2), axis=0, keepdims=True)
  out_ref[...] = jnp.sum(counts * lse, axis=1, keepdims=True) * (1.0 / _B)


_combine = pl.pallas_call(
    _combine_body,
    out_shape=jax.ShapeDtypeStruct((1, 1), jnp.float32),
)


@jax.jit
def kernel(weight, target):
  partials = _hist()(target)
  lse = _lse(weight)
  out = _combine(lse, partials)
  return out[0, 0]
